# Initial kernel scaffold; baseline (speedup 1.0000x reference)
import jax
import jax.numpy as jnp
from jax.experimental import pallas as pl


def kernel(h, edges, coords, W0, b0, W1, b1, W2, b2):
    raise NotImplementedError("write your pallas kernel here")


# speedup vs baseline: 6.2889x; 6.2889x over previous
import functools

import jax
import jax.numpy as jnp
from jax import lax
from jax.experimental import pallas as pl
from jax.experimental.pallas import tpu as pltpu
from jax.experimental.pallas import tpu_sc as plsc

N = 10000
NPAD = 10240
F = 128
KIN_PAD = 256
NC, NS = 2, 16
NW = NC * NS
CH = 128
HALF = NPAD // 2
RPT = HALF // NS
ZCH = 64
NZ = RPT // ZCH
RBLK = 512
GRID = NPAD // RBLK

_mesh = plsc.VectorSubcoreMesh(
    core_axis_name="c", subcore_axis_name="s", num_cores=NC, num_subcores=NS)


DW = 16


def _deg_body(dst_hbm, out_hbm, idx_d, buf, acc16, sem0):
  kch = idx_d.shape[0]
  cid = lax.axis_index("c")
  sid = lax.axis_index("s")
  base = sid * RPT

  def fill(val):
    def body(r, _):
      buf[r, pl.ds(0, DW)] = jnp.full((DW,), val, jnp.float32)
      return _
    lax.fori_loop(0, CH, body, None)

  fill(0.0)
  for r in range(NZ):
    pltpu.sync_copy(buf.at[pl.ds(0, ZCH)],
                    acc16.at[pl.ds(base + r * ZCH, ZCH)])
  pltpu.sync_copy(dst_hbm.at[sid], idx_d)

  off = cid * HALF

  def remap_row(j, _):
    def remap_grp(g, __):
      d = idx_d[j, pl.ds(g * 16, 16)] - off
      bad = (d < 0) | (d >= HALF)
      idx_d[j, pl.ds(g * 16, 16)] = jnp.where(bad, HALF, d)
      return __
    lax.fori_loop(0, CH // 16, remap_grp, None)
    return _
  lax.fori_loop(0, kch, remap_row, None)
  fill(1.0)
  plsc.subcore_barrier()

  def step(j, _):
    pltpu.sync_copy(buf, acc16.at[idx_d.at[j]], add=True)
    return _
  lax.fori_loop(0, kch, step, None)
  plsc.subcore_barrier()

  for r in range(NZ):
    pltpu.sync_copy(acc16.at[pl.ds(base + r * ZCH, ZCH)],
                    buf.at[pl.ds(0, ZCH)])
    pltpu.sync_copy(buf.at[pl.ds(0, ZCH)],
                    out_hbm.at[pl.ds(off + base + r * ZCH, ZCH)])


def _make_deg_kernel(kch):
  return pl.kernel(
      _deg_body,
      out_type=jax.ShapeDtypeStruct((NPAD, DW), jnp.float32),
      mesh=_mesh,
      scratch_types=[
          pltpu.VMEM((kch, CH), jnp.int32),
          pltpu.VMEM((CH, DW), jnp.float32),
          pltpu.VMEM_SHARED((HALF + CH, DW), jnp.float32),
          pltpu.SemaphoreType.DMA,
      ],
  )


def _prop_body(ys_hbm, src_hbm, dst_hbm, zrow_hbm, out_hbm,
               idx_s, idx_d, rows0, rows1, acc_sh, sem0, sem1):
  kch = idx_s.shape[0]
  cid = lax.axis_index("c")
  sid = lax.axis_index("s")
  base = sid * RPT

  pltpu.sync_copy(zrow_hbm, rows0)
  for r in range(NZ):
    pltpu.sync_copy(rows0.at[pl.ds(0, ZCH)],
                    acc_sh.at[pl.ds(base + r * ZCH, ZCH)])
  pltpu.sync_copy(src_hbm.at[sid], idx_s)
  pltpu.sync_copy(dst_hbm.at[sid], idx_d)

  off = cid * HALF

  def remap_row(j, _):
    def remap_grp(g, __):
      d = idx_d[j, pl.ds(g * 16, 16)] - off
      bad = (d < 0) | (d >= HALF)
      idx_d[j, pl.ds(g * 16, 16)] = jnp.where(bad, HALF, d)
      return __
    lax.fori_loop(0, CH // 16, remap_grp, None)
    return _
  lax.fori_loop(0, kch, remap_row, None)
  plsc.subcore_barrier()

  pltpu.async_copy(ys_hbm.at[idx_s.at[0]], rows0, sem0)

  def step(i, _):
    j0 = 2 * i
    j1 = j0 + 1
    pltpu.async_copy(ys_hbm.at[idx_s.at[j1]], rows1, sem1)
    pltpu.make_async_copy(ys_hbm.at[idx_s.at[j0]], rows0, sem0).wait()
    pltpu.sync_copy(rows0, acc_sh.at[idx_d.at[j0]], add=True)

    @pl.when(j0 + 2 < kch)
    def _():
      pltpu.async_copy(ys_hbm.at[idx_s.at[j0 + 2]], rows0, sem0)

    pltpu.make_async_copy(ys_hbm.at[idx_s.at[j1]], rows1, sem1).wait()
    pltpu.sync_copy(rows1, acc_sh.at[idx_d.at[j1]], add=True)
    return _

  lax.fori_loop(0, kch // 2, step, None)
  plsc.subcore_barrier()

  for r in range(NZ):
    pltpu.sync_copy(acc_sh.at[pl.ds(base + r * ZCH, ZCH)],
                    rows0.at[pl.ds(0, ZCH)])
    pltpu.sync_copy(rows0.at[pl.ds(0, ZCH)],
                    out_hbm.at[pl.ds(off + base + r * ZCH, ZCH)])


def _make_prop_kernel(kch):
  return pl.kernel(
      _prop_body,
      out_type=jax.ShapeDtypeStruct((NPAD, F), jnp.float32),
      mesh=_mesh,
      scratch_types=[
          pltpu.VMEM((kch, CH), jnp.int32),
          pltpu.VMEM((kch, CH), jnp.int32),
          pltpu.VMEM((CH, F), jnp.float32),
          pltpu.VMEM((CH, F), jnp.float32),
          pltpu.VMEM_SHARED((HALF + CH, F), jnp.float32),
          pltpu.SemaphoreType.DMA,
          pltpu.SemaphoreType.DMA,
      ],
  )


def _mm0_body(x_ref, w_ref, degt_ref, ys_ref, dinv_ref):
  deg = degt_ref[...][:, 0:1] + 1.0
  dinv = lax.rsqrt(deg)
  xw = jnp.dot(x_ref[...], w_ref[...], preferred_element_type=jnp.float32)
  ys_ref[...] = xw * dinv
  dinv_ref[...] = dinv


def _mm0(x_pad, w0p, deg_t):
  return pl.pallas_call(
      _mm0_body,
      grid=(GRID,),
      in_specs=[
          pl.BlockSpec((RBLK, KIN_PAD), lambda i: (i, 0)),
          pl.BlockSpec((KIN_PAD, F), lambda i: (0, 0)),
          pl.BlockSpec((RBLK, DW), lambda i: (i, 0)),
      ],
      out_specs=[
          pl.BlockSpec((RBLK, F), lambda i: (i, 0)),
          pl.BlockSpec((RBLK, 1), lambda i: (i, 0)),
      ],
      out_shape=[
          jax.ShapeDtypeStruct((NPAD, F), jnp.float32),
          jax.ShapeDtypeStruct((NPAD, 1), jnp.float32),
      ],
  )(x_pad, w0p, deg_t)


def _layer_body(acc_ref, ys_ref, dinv_ref, b_ref, w_ref, out_ref):
  t = acc_ref[...] + ys_ref[...]
  dinv = dinv_ref[...]
  x = jnp.maximum(t * dinv + b_ref[...], 0.0)
  out_ref[...] = jnp.dot(
      x, w_ref[...], preferred_element_type=jnp.float32) * dinv


def _layer(acc, ys, dinv, b, w):
  return pl.pallas_call(
      _layer_body,
      grid=(GRID,),
      in_specs=[
          pl.BlockSpec((RBLK, F), lambda i: (i, 0)),
          pl.BlockSpec((RBLK, F), lambda i: (i, 0)),
          pl.BlockSpec((RBLK, 1), lambda i: (i, 0)),
          pl.BlockSpec((1, F), lambda i: (0, 0)),
          pl.BlockSpec((F, F), lambda i: (0, 0)),
      ],
      out_specs=pl.BlockSpec((RBLK, F), lambda i: (i, 0)),
      out_shape=jax.ShapeDtypeStruct((NPAD, F), jnp.float32),
  )(acc, ys, dinv, b, w)


def _final_body(acc_ref, ys_ref, dinv_ref, b_ref, out_ref):
  t = acc_ref[...] + ys_ref[...]
  out_ref[...] = jnp.maximum(t * dinv_ref[...] + b_ref[...], 0.0)


def _final(acc, ys, dinv, b):
  return pl.pallas_call(
      _final_body,
      grid=(GRID,),
      in_specs=[
          pl.BlockSpec((RBLK, F), lambda i: (i, 0)),
          pl.BlockSpec((RBLK, F), lambda i: (i, 0)),
          pl.BlockSpec((RBLK, 1), lambda i: (i, 0)),
          pl.BlockSpec((1, F), lambda i: (0, 0)),
      ],
      out_specs=pl.BlockSpec((RBLK, F), lambda i: (i, 0)),
      out_shape=jax.ShapeDtypeStruct((NPAD, F), jnp.float32),
  )(acc, ys, dinv, b)


@jax.jit
def kernel(h, edges, coords, W0, b0, W1, b1, W2, b2):
  e = edges.shape[1]
  kch = 2 * -(-e // (2 * NS * CH))
  epad = NS * kch * CH
  src_p = jnp.concatenate(
      [edges[0], jnp.full((epad - e,), N, jnp.int32)])
  dst_p = jnp.concatenate(
      [edges[1], jnp.full((epad - e,), N, jnp.int32)])
  src3 = src_p.reshape(NS, kch, CH)
  dst3 = dst_p.reshape(NS, kch, CH)

  x_in = jnp.concatenate([h[0, 0], coords[0, 0]], axis=1)
  x_pad = jnp.pad(x_in, ((0, NPAD - N), (0, KIN_PAD - x_in.shape[1])))
  w0p = jnp.pad(W0, ((0, KIN_PAD - W0.shape[0]), (0, 0)))
  zrow = jnp.zeros((CH, F), jnp.float32)

  deg16 = _make_deg_kernel(kch)(dst3)

  ys0, dinv = _mm0(x_pad, w0p, deg16)
  prop = _make_prop_kernel(kch)

  acc = prop(ys0, src3, dst3, zrow)
  ys1 = _layer(acc, ys0, dinv, b0.reshape(1, F), W1)
  acc = prop(ys1, src3, dst3, zrow)
  ys2 = _layer(acc, ys1, dinv, b1.reshape(1, F), W2)
  acc = prop(ys2, src3, dst3, zrow)
  xf = _final(acc, ys2, dinv, b2.reshape(1, F))
  return xf[:N].reshape(1, 1, N, F)
